# baseline (speedup 1.0000x reference)
import functools

import jax
import jax.numpy as jnp
from jax import lax
from jax.experimental import pallas as pl
from jax.experimental.pallas import tpu as pltpu
from jax.experimental.pallas import tpu_sc as plsc

N = 4096
E = 65536
F = 256
OUT = 128
KARY = 5

NC = 2
NS = 16
L = 16
HALF = N // NC
NPT = HALF // NS
CHUNK = E // NS


def _sc_build_body(adj_hbm, selT_hbm, len_hbm,
                   adj_v, cnt_v, loc_v, cnt_t, loc_t, selT, len_t,
                   counts_sh, locs_sh):
    cid = lax.axis_index("c")
    sid = lax.axis_index("s")
    lo = cid * HALF
    lanes = lax.iota(jnp.int32, L)

    pltpu.sync_copy(adj_hbm.at[pl.ds(sid * CHUNK * 2, CHUNK * 2)], adj_v)

    @plsc.parallel_loop(0, HALF, step=L)
    def _(i):
        cnt_v[pl.ds(i, L)] = jnp.zeros((L,), jnp.int32)

    def edge_body(i, _):
        idx2 = (i * L + lanes) * 2
        s16 = plsc.load_gather(adj_v, [idx2])
        d16 = plsc.load_gather(adj_v, [idx2 + 1])
        sl_raw = s16 - lo
        valid = (sl_raw >= 0) & (sl_raw < HALF)
        occ, lastm = plsc.scan_count(sl_raw, mask=valid)
        sl = sl_raw & (HALF - 1)
        cnt16 = plsc.load_gather(cnt_v, [sl], mask=valid)
        p = cnt16 + occ - 1
        plsc.store_scatter(loc_v, [sl * KARY + p], d16,
                           mask=valid & (p < KARY))
        plsc.store_scatter(cnt_v, [sl], cnt16 + occ, mask=valid & lastm)
        return 0

    lax.fori_loop(0, CHUNK // L, edge_body, 0)

    pltpu.sync_copy(cnt_v, counts_sh.at[sid])
    pltpu.sync_copy(loc_v, locs_sh.at[sid])
    plsc.subcore_barrier()

    nb = sid * NPT
    pltpu.sync_copy(counts_sh.at[:, pl.ds(nb, NPT)], cnt_t)
    pltpu.sync_copy(locs_sh.at[:, pl.ds(nb * KARY, NPT * KARY)], loc_t)

    @plsc.parallel_loop(0, NPT * KARY, step=L)
    def _(i):
        selT[pl.ds(i, L)] = jnp.zeros((L,), jnp.int32)

    @plsc.parallel_loop(0, NPT, step=L)
    def _(j):
        acc = jnp.zeros((L,), jnp.int32)
        for w in range(NS):
            acc = acc + jnp.minimum(cnt_t[w, pl.ds(j, L)], KARY)
        len_t[pl.ds(j, L)] = jnp.minimum(acc, KARY)

    def merge_body(j, _):
        jv = jnp.full((L,), j, jnp.int32)
        col = plsc.load_gather(cnt_t, [lanes, jv])
        cc = jnp.minimum(col, KARY)
        off = plsc.cumsum(cc) - cc
        for k in range(KARY):
            vals = plsc.load_gather(loc_t, [lanes, jv * KARY + k])
            pos = off + k
            m = (k < cc) & (pos < KARY)
            plsc.store_scatter(selT, [pos * NPT + j], vals, mask=m)
        return 0

    lax.fori_loop(0, NPT, merge_body, 0)

    pltpu.sync_copy(len_t, len_hbm.at[pl.ds(lo + nb, NPT)])
    for t in range(KARY):
        pltpu.sync_copy(selT.at[pl.ds(t * NPT, NPT)],
                        selT_hbm.at[pl.ds(t * N + lo + nb, NPT)])


def _sc_build(adj_flat):
    mesh = plsc.VectorSubcoreMesh(core_axis_name="c", subcore_axis_name="s")
    kern = pl.kernel(
        _sc_build_body,
        out_type=[
            jax.ShapeDtypeStruct((KARY * N,), jnp.int32),
            jax.ShapeDtypeStruct((N,), jnp.int32),
        ],
        mesh=mesh,
        scratch_types=[
            pltpu.VMEM((CHUNK * 2,), jnp.int32),
            pltpu.VMEM((HALF,), jnp.int32),
            pltpu.VMEM((HALF * KARY,), jnp.int32),
            pltpu.VMEM((NS, NPT), jnp.int32),
            pltpu.VMEM((NS, NPT * KARY), jnp.int32),
            pltpu.VMEM((NPT * KARY,), jnp.int32),
            pltpu.VMEM((NPT,), jnp.int32),
            pltpu.VMEM_SHARED((NS, HALF), jnp.int32),
            pltpu.VMEM_SHARED((NS, HALF * KARY), jnp.int32),
        ],
        compiler_params=pltpu.CompilerParams(needs_layout_passes=False),
    )
    return kern(adj_flat)


SPLIT = 4
NW = NC * NS
NPQ = (N // SPLIT) // NW


def _sc_gather_body(q, x_hbm, selT_hbm, feats_hbm, sel_v, rows_v, sem):
    cid = lax.axis_index("c")
    sid = lax.axis_index("s")
    wid = sid * NC + cid
    gbase = q * (N // SPLIT) + wid * NPQ
    obase = wid * NPQ
    for t in range(KARY):
        pltpu.sync_copy(selT_hbm.at[pl.ds(t * N + gbase, NPQ)], sel_v)
        pltpu.async_copy(x_hbm.at[sel_v], rows_v, sem).wait()
        pltpu.sync_copy(rows_v, feats_hbm.at[t, pl.ds(obase, NPQ)])


def _sc_gather(x_pk, selT, q):
    mesh = plsc.VectorSubcoreMesh(core_axis_name="c", subcore_axis_name="s")
    kern = pl.kernel(
        functools.partial(_sc_gather_body, q),
        out_type=jax.ShapeDtypeStruct((KARY, N // SPLIT, F // 2), jnp.int32),
        mesh=mesh,
        scratch_types=[
            pltpu.VMEM((NPQ,), jnp.int32),
            pltpu.VMEM((NPQ, F // 2), jnp.int32),
            pltpu.SemaphoreType.DMA,
        ],
        compiler_params=pltpu.CompilerParams(needs_layout_passes=False),
    )
    return kern(x_pk, selT)


def _pack_body(x_ref, wih_ref, whh_ref, wout_ref, xpk_ref, wihb_ref, whhb_ref,
               woutb_ref):
    xa = lax.bitcast_convert_type(x_ref[:, :F // 2], jnp.int32)
    xb = lax.bitcast_convert_type(x_ref[:, F // 2:], jnp.int32)
    half = jnp.int32(0x8000)
    lo = ((xa + half) >> 16) & jnp.int32(0xFFFF)
    hi = (xb + half) & jnp.int32(-65536)
    xpk_ref[...] = lo | hi
    bf = jnp.bfloat16
    wihb_ref[...] = wih_ref[...].astype(bf)
    whhb_ref[...] = whh_ref[...].astype(bf)
    woutb_ref[...] = wout_ref[...].astype(bf)


def _tc_pack(x, W_ih, W_hh, W_out):
    bf = jnp.bfloat16
    return pl.pallas_call(
        _pack_body,
        out_shape=[
            jax.ShapeDtypeStruct((N, F // 2), jnp.int32),
            jax.ShapeDtypeStruct((4 * F, F), bf),
            jax.ShapeDtypeStruct((4 * F, F), bf),
            jax.ShapeDtypeStruct((2 * F, OUT), bf),
        ],
    )(x, W_ih, W_hh, W_out)


def _dot_t(a, w):
    return lax.dot_general(a, w, (((1,), (1,)), ((), ())),
                           preferred_element_type=jnp.float32)


def _tc_body(x_ref, feats_ref, len_ref, wih_ref, whh_ref, bih_ref,
             bhh_ref, wout_ref, bout_ref, out_ref):
    xb = x_ref[...]
    b = bih_ref[...] + bhh_ref[...]
    lens = len_ref[...]
    blk = xb.shape[0]
    c = jnp.zeros((blk, F), jnp.float32)
    h = jnp.zeros((blk, F), jnp.float32)
    bf = jnp.bfloat16
    for t in range(KARY):
        fi = feats_ref[t]
        ft_lo = lax.bitcast_convert_type(fi << 16, jnp.float32).astype(bf)
        ft_hi = lax.bitcast_convert_type(fi & jnp.int32(-65536),
                                         jnp.float32).astype(bf)
        gates = (_dot_t(ft_lo, wih_ref[:, :F // 2])
                 + _dot_t(ft_hi, wih_ref[:, F // 2:]) + b)
        if t > 0:
            gates = gates + _dot_t(h.astype(bf), whh_ref[...])
        i_g = gates[:, :F]
        f_g = gates[:, F:2 * F]
        g_g = gates[:, 2 * F:3 * F]
        o_g = gates[:, 3 * F:]
        c_new = jax.nn.sigmoid(f_g) * c + jax.nn.sigmoid(i_g) * jnp.tanh(g_g)
        h_new = jax.nn.sigmoid(o_g) * jnp.tanh(c_new)
        m = t < lens
        c = jnp.where(m, c_new, c)
        h = jnp.where(m, h_new, h)
    y = (jnp.dot(xb.astype(bf), wout_ref[:F],
                 preferred_element_type=jnp.float32)
         + jnp.dot(c.astype(bf), wout_ref[F:],
                   preferred_element_type=jnp.float32)
         + bout_ref[...])
    out_ref[...] = jax.nn.sigmoid(y)


def _tc_lstm(x, feats, lens, wihb, whhb, bih, bhh, woutb, bout, q):
    B = 512
    hb = q * ((N // SPLIT) // B)
    grid = ((N // SPLIT) // B,)
    return pl.pallas_call(
        _tc_body,
        grid=grid,
        in_specs=[
            pl.BlockSpec((B, F), lambda i: (i + hb, 0)),
            pl.BlockSpec((KARY, B, F // 2), lambda i: (0, i, 0)),
            pl.BlockSpec((B, 1), lambda i: (i + hb, 0)),
            pl.BlockSpec((4 * F, F), lambda i: (0, 0)),
            pl.BlockSpec((4 * F, F), lambda i: (0, 0)),
            pl.BlockSpec((1, 4 * F), lambda i: (0, 0)),
            pl.BlockSpec((1, 4 * F), lambda i: (0, 0)),
            pl.BlockSpec((2 * F, OUT), lambda i: (0, 0)),
            pl.BlockSpec((1, OUT), lambda i: (0, 0)),
        ],
        out_specs=pl.BlockSpec((B, OUT), lambda i: (i, 0)),
        out_shape=jax.ShapeDtypeStruct((N // SPLIT, OUT), jnp.float32),
        compiler_params=pltpu.CompilerParams(
            dimension_semantics=("arbitrary",)),
    )(x, feats, lens, wihb, whhb, bih, bhh, woutb, bout)


def kernel(node_feat_input, adjacency_input, indices, W_ih, W_hh, b_ih, b_hh,
           W_out, b_out):
    x_pk, wihb, whhb, woutb = _tc_pack(node_feat_input, W_ih, W_hh, W_out)
    selT, lengths = _sc_build(adjacency_input.reshape(2 * E))
    lens2 = lengths.reshape(N, 1)
    bih2 = b_ih.reshape(1, -1)
    bhh2 = b_hh.reshape(1, -1)
    bout2 = b_out.reshape(1, -1)
    outs = []
    for q in range(SPLIT):
        feats_q = _sc_gather(x_pk, selT, q)
        outs.append(_tc_lstm(node_feat_input, feats_q, lens2, wihb, whhb,
                             bih2, bhh2, woutb, bout2, q))
    return jnp.concatenate(outs, axis=0)


# speedup vs baseline: 1.3906x; 1.3906x over previous
import functools

import jax
import jax.numpy as jnp
from jax import lax
from jax.experimental import pallas as pl
from jax.experimental.pallas import tpu as pltpu
from jax.experimental.pallas import tpu_sc as plsc

N = 4096
E = 65536
F = 256
OUT = 128
KARY = 5

NC = 2
NS = 16
L = 16
HALF = N // NC
NPT = HALF // NS
CHUNK = E // NS


def _sc_build_body(src_hbm, dst_hbm, selT_hbm, len_hbm,
                   src_v, dst_v, cnt_v, loc_v, cnt_t, loc_t, selT, len_t,
                   counts_sh, locs_sh):
    cid = lax.axis_index("c")
    sid = lax.axis_index("s")
    lo = cid * HALF
    lanes = lax.iota(jnp.int32, L)

    pltpu.sync_copy(src_hbm.at[pl.ds(sid * CHUNK, CHUNK)], src_v)
    pltpu.sync_copy(dst_hbm.at[pl.ds(sid * CHUNK, CHUNK)], dst_v)

    @plsc.parallel_loop(0, HALF, step=L)
    def _(i):
        cnt_v[pl.ds(i, L)] = jnp.zeros((L,), jnp.int32)

    def edge_body(i, _):
        s16 = src_v[pl.ds(i * L, L)]
        d16 = dst_v[pl.ds(i * L, L)]
        sl_raw = s16 - lo
        valid = (sl_raw >= 0) & (sl_raw < HALF)
        occ, lastm = plsc.scan_count(sl_raw, mask=valid)
        sl = sl_raw & (HALF - 1)
        cnt16 = plsc.load_gather(cnt_v, [sl], mask=valid)
        p = cnt16 + occ - 1
        plsc.store_scatter(loc_v, [sl * KARY + p], d16,
                           mask=valid & (p < KARY))
        plsc.store_scatter(cnt_v, [sl], cnt16 + occ, mask=valid & lastm)
        return 0

    lax.fori_loop(0, CHUNK // L, edge_body, 0)

    pltpu.sync_copy(cnt_v, counts_sh.at[sid])
    pltpu.sync_copy(loc_v, locs_sh.at[sid])
    plsc.subcore_barrier()

    nb = sid * NPT
    pltpu.sync_copy(counts_sh.at[:, pl.ds(nb, NPT)], cnt_t)
    pltpu.sync_copy(locs_sh.at[:, pl.ds(nb * KARY, NPT * KARY)], loc_t)

    @plsc.parallel_loop(0, NPT * KARY, step=L)
    def _(i):
        selT[pl.ds(i, L)] = jnp.zeros((L,), jnp.int32)

    @plsc.parallel_loop(0, NPT, step=L)
    def _(j):
        acc = jnp.zeros((L,), jnp.int32)
        for w in range(NS):
            acc = acc + jnp.minimum(cnt_t[w, pl.ds(j, L)], KARY)
        len_t[pl.ds(j, L)] = jnp.minimum(acc, KARY)

    def merge_body(j, _):
        jv = jnp.full((L,), j, jnp.int32)
        col = plsc.load_gather(cnt_t, [lanes, jv])
        cc = jnp.minimum(col, KARY)
        off = plsc.cumsum(cc) - cc
        for k in range(KARY):
            vals = plsc.load_gather(loc_t, [lanes, jv * KARY + k])
            pos = off + k
            m = (k < cc) & (pos < KARY)
            plsc.store_scatter(selT, [pos * NPT + j], vals, mask=m)
        return 0

    lax.fori_loop(0, NPT, merge_body, 0)

    pltpu.sync_copy(len_t, len_hbm.at[pl.ds(lo + nb, NPT)])
    for t in range(KARY):
        pltpu.sync_copy(selT.at[pl.ds(t * NPT, NPT)],
                        selT_hbm.at[pl.ds(t * N + lo + nb, NPT)])


def _sc_build(src, dst):
    mesh = plsc.VectorSubcoreMesh(core_axis_name="c", subcore_axis_name="s")
    kern = pl.kernel(
        _sc_build_body,
        out_type=[
            jax.ShapeDtypeStruct((KARY * N,), jnp.int32),
            jax.ShapeDtypeStruct((N,), jnp.int32),
        ],
        mesh=mesh,
        scratch_types=[
            pltpu.VMEM((CHUNK,), jnp.int32),
            pltpu.VMEM((CHUNK,), jnp.int32),
            pltpu.VMEM((HALF,), jnp.int32),
            pltpu.VMEM((HALF * KARY,), jnp.int32),
            pltpu.VMEM((NS, NPT), jnp.int32),
            pltpu.VMEM((NS, NPT * KARY), jnp.int32),
            pltpu.VMEM((NPT * KARY,), jnp.int32),
            pltpu.VMEM((NPT,), jnp.int32),
            pltpu.VMEM_SHARED((NS, HALF), jnp.int32),
            pltpu.VMEM_SHARED((NS, HALF * KARY), jnp.int32),
        ],
        compiler_params=pltpu.CompilerParams(needs_layout_passes=False),
    )
    return kern(src, dst)


STAGES = (1024, 1536, 1536)
NW = NC * NS


def _sc_gather_body(base, npq, x_hbm, selT_hbm, feats_hbm, sel_v, rows_v,
                    sem):
    cid = lax.axis_index("c")
    sid = lax.axis_index("s")
    wid = sid * NC + cid
    gbase = base + wid * npq
    obase = wid * npq
    for t in range(KARY):
        pltpu.sync_copy(selT_hbm.at[pl.ds(t * N + gbase, npq)], sel_v)
        pltpu.async_copy(x_hbm.at[sel_v], rows_v, sem).wait()
        pltpu.sync_copy(rows_v, feats_hbm.at[t, pl.ds(obase, npq)])


def _sc_gather(x_pk, selT, base, count):
    npq = count // NW
    mesh = plsc.VectorSubcoreMesh(core_axis_name="c", subcore_axis_name="s")
    kern = pl.kernel(
        functools.partial(_sc_gather_body, base, npq),
        out_type=jax.ShapeDtypeStruct((KARY, count, F // 2), jnp.int32),
        mesh=mesh,
        scratch_types=[
            pltpu.VMEM((npq,), jnp.int32),
            pltpu.VMEM((npq, F // 2), jnp.int32),
            pltpu.SemaphoreType.DMA,
        ],
        compiler_params=pltpu.CompilerParams(needs_layout_passes=False),
    )
    return kern(x_pk, selT)


def _pack_body(x_ref, wih_ref, whh_ref, wout_ref, xpk_ref, wihb_ref, whhb_ref,
               woutb_ref):
    xa = lax.bitcast_convert_type(x_ref[:, :F // 2], jnp.int32)
    xb = lax.bitcast_convert_type(x_ref[:, F // 2:], jnp.int32)
    half = jnp.int32(0x8000)
    lo = ((xa + half) >> 16) & jnp.int32(0xFFFF)
    hi = (xb + half) & jnp.int32(-65536)
    xpk_ref[...] = lo | hi
    bf = jnp.bfloat16
    wihb_ref[...] = wih_ref[...].astype(bf)
    whhb_ref[...] = whh_ref[...].astype(bf)
    woutb_ref[...] = wout_ref[...].astype(bf)


def _tc_pack(x, W_ih, W_hh, W_out):
    bf = jnp.bfloat16
    return pl.pallas_call(
        _pack_body,
        out_shape=[
            jax.ShapeDtypeStruct((N, F // 2), jnp.int32),
            jax.ShapeDtypeStruct((4 * F, F), bf),
            jax.ShapeDtypeStruct((4 * F, F), bf),
            jax.ShapeDtypeStruct((2 * F, OUT), bf),
        ],
    )(x, W_ih, W_hh, W_out)


def _dot_t(a, w):
    return lax.dot_general(a, w, (((1,), (1,)), ((), ())),
                           preferred_element_type=jnp.float32)


def _tc_body(x_ref, feats_ref, len_ref, wih_ref, whh_ref, bih_ref,
             bhh_ref, wout_ref, bout_ref, out_ref):
    xb = x_ref[...]
    b = bih_ref[...] + bhh_ref[...]
    lens = len_ref[...]
    blk = xb.shape[0]
    c = jnp.zeros((blk, F), jnp.float32)
    h = jnp.zeros((blk, F), jnp.float32)
    bf = jnp.bfloat16
    for t in range(KARY):
        fi = feats_ref[t]
        ft_lo = lax.bitcast_convert_type(fi << 16, jnp.float32).astype(bf)
        ft_hi = lax.bitcast_convert_type(fi & jnp.int32(-65536),
                                         jnp.float32).astype(bf)
        gates = (_dot_t(ft_lo, wih_ref[:, :F // 2])
                 + _dot_t(ft_hi, wih_ref[:, F // 2:]) + b)
        if t > 0:
            gates = gates + _dot_t(h.astype(bf), whh_ref[...])
        i_g = gates[:, :F]
        f_g = gates[:, F:2 * F]
        g_g = gates[:, 2 * F:3 * F]
        o_g = gates[:, 3 * F:]
        c_new = jax.nn.sigmoid(f_g) * c + jax.nn.sigmoid(i_g) * jnp.tanh(g_g)
        h_new = jax.nn.sigmoid(o_g) * jnp.tanh(c_new)
        m = t < lens
        c = jnp.where(m, c_new, c)
        h = jnp.where(m, h_new, h)
    y = (jnp.dot(xb.astype(bf), wout_ref[:F],
                 preferred_element_type=jnp.float32)
         + jnp.dot(c.astype(bf), wout_ref[F:],
                   preferred_element_type=jnp.float32)
         + bout_ref[...])
    out_ref[...] = jax.nn.sigmoid(y)


def _tc_lstm(x, feats, lens, wihb, whhb, bih, bhh, woutb, bout, base, count):
    B = 512
    hb = base // B
    grid = (count // B,)
    return pl.pallas_call(
        _tc_body,
        grid=grid,
        in_specs=[
            pl.BlockSpec((B, F), lambda i: (i + hb, 0)),
            pl.BlockSpec((KARY, B, F // 2), lambda i: (0, i, 0)),
            pl.BlockSpec((B, 1), lambda i: (i + hb, 0)),
            pl.BlockSpec((4 * F, F), lambda i: (0, 0)),
            pl.BlockSpec((4 * F, F), lambda i: (0, 0)),
            pl.BlockSpec((1, 4 * F), lambda i: (0, 0)),
            pl.BlockSpec((1, 4 * F), lambda i: (0, 0)),
            pl.BlockSpec((2 * F, OUT), lambda i: (0, 0)),
            pl.BlockSpec((1, OUT), lambda i: (0, 0)),
        ],
        out_specs=pl.BlockSpec((B, OUT), lambda i: (i, 0)),
        out_shape=jax.ShapeDtypeStruct((count, OUT), jnp.float32),
        compiler_params=pltpu.CompilerParams(
            dimension_semantics=("arbitrary",)),
    )(x, feats, lens, wihb, whhb, bih, bhh, woutb, bout)


def kernel(node_feat_input, adjacency_input, indices, W_ih, W_hh, b_ih, b_hh,
           W_out, b_out):
    src = adjacency_input[:, 0]
    dst = adjacency_input[:, 1]
    x_pk, wihb, whhb, woutb = _tc_pack(node_feat_input, W_ih, W_hh, W_out)
    selT, lengths = _sc_build(src, dst)
    lens2 = lengths.reshape(N, 1)
    bih2 = b_ih.reshape(1, -1)
    bhh2 = b_hh.reshape(1, -1)
    bout2 = b_out.reshape(1, -1)
    outs = []
    base = 0
    for count in STAGES:
        feats_q = _sc_gather(x_pk, selT, base, count)
        outs.append(_tc_lstm(node_feat_input, feats_q, lens2, wihb, whhb,
                             bih2, bhh2, woutb, bout2, base, count))
        base += count
    return jnp.concatenate(outs, axis=0)


# speedup vs baseline: 1.5084x; 1.0847x over previous
import functools

import jax
import jax.numpy as jnp
from jax import lax
from jax.experimental import pallas as pl
from jax.experimental.pallas import tpu as pltpu
from jax.experimental.pallas import tpu_sc as plsc

N = 4096
E = 65536
F = 256
OUT = 128
KARY = 5

NC = 2
NS = 16
L = 16
HALF = N // NC
NPT = HALF // NS
CHUNK = E // NS


def _sc_build_body(src_hbm, dst_hbm, selT_hbm, len_hbm,
                   src_v, dst_v, cnt_v, loc_v, cnt_t, loc_t, selT, len_t,
                   counts_sh, locs_sh):
    cid = lax.axis_index("c")
    sid = lax.axis_index("s")
    lo = cid * HALF
    lanes = lax.iota(jnp.int32, L)

    pltpu.sync_copy(src_hbm.at[pl.ds(sid * CHUNK, CHUNK)], src_v)
    pltpu.sync_copy(dst_hbm.at[pl.ds(sid * CHUNK, CHUNK)], dst_v)

    @plsc.parallel_loop(0, HALF, step=L)
    def _(i):
        cnt_v[pl.ds(i, L)] = jnp.zeros((L,), jnp.int32)

    def edge_body(i, _):
        s16 = src_v[pl.ds(i * L, L)]
        d16 = dst_v[pl.ds(i * L, L)]
        sl_raw = s16 - lo
        valid = (sl_raw >= 0) & (sl_raw < HALF)
        occ, lastm = plsc.scan_count(sl_raw, mask=valid)
        sl = sl_raw & (HALF - 1)
        cnt16 = plsc.load_gather(cnt_v, [sl], mask=valid)
        p = cnt16 + occ - 1
        plsc.store_scatter(loc_v, [sl * KARY + p], d16,
                           mask=valid & (p < KARY))
        plsc.store_scatter(cnt_v, [sl], cnt16 + occ, mask=valid & lastm)
        return 0

    lax.fori_loop(0, CHUNK // L, edge_body, 0)

    pltpu.sync_copy(cnt_v, counts_sh.at[sid])
    pltpu.sync_copy(loc_v, locs_sh.at[sid])
    plsc.subcore_barrier()

    nb = sid * NPT
    pltpu.sync_copy(counts_sh.at[:, pl.ds(nb, NPT)], cnt_t)
    pltpu.sync_copy(locs_sh.at[:, pl.ds(nb * KARY, NPT * KARY)], loc_t)

    @plsc.parallel_loop(0, NPT * KARY, step=L)
    def _(i):
        selT[pl.ds(i, L)] = jnp.zeros((L,), jnp.int32)

    @plsc.parallel_loop(0, NPT, step=L)
    def _(j):
        acc = jnp.zeros((L,), jnp.int32)
        for w in range(NS):
            acc = acc + jnp.minimum(cnt_t[w, pl.ds(j, L)], KARY)
        len_t[pl.ds(j, L)] = jnp.minimum(acc, KARY)

    def merge_body(j, _):
        jv = jnp.full((L,), j, jnp.int32)
        col = plsc.load_gather(cnt_t, [lanes, jv])
        cc = jnp.minimum(col, KARY)
        off = plsc.cumsum(cc) - cc
        for k in range(KARY):
            vals = plsc.load_gather(loc_t, [lanes, jv * KARY + k])
            pos = off + k
            m = (k < cc) & (pos < KARY)
            plsc.store_scatter(selT, [pos * NPT + j], vals, mask=m)
        return 0

    lax.fori_loop(0, NPT, merge_body, 0)

    pltpu.sync_copy(len_t, len_hbm.at[pl.ds(lo + nb, NPT)])
    for t in range(KARY):
        pltpu.sync_copy(selT.at[pl.ds(t * NPT, NPT)],
                        selT_hbm.at[pl.ds(t * N + lo + nb, NPT)])


def _sc_build(src, dst):
    mesh = plsc.VectorSubcoreMesh(core_axis_name="c", subcore_axis_name="s")
    kern = pl.kernel(
        _sc_build_body,
        out_type=[
            jax.ShapeDtypeStruct((KARY * N,), jnp.int32),
            jax.ShapeDtypeStruct((N,), jnp.int32),
        ],
        mesh=mesh,
        scratch_types=[
            pltpu.VMEM((CHUNK,), jnp.int32),
            pltpu.VMEM((CHUNK,), jnp.int32),
            pltpu.VMEM((HALF,), jnp.int32),
            pltpu.VMEM((HALF * KARY,), jnp.int32),
            pltpu.VMEM((NS, NPT), jnp.int32),
            pltpu.VMEM((NS, NPT * KARY), jnp.int32),
            pltpu.VMEM((NPT * KARY,), jnp.int32),
            pltpu.VMEM((NPT,), jnp.int32),
            pltpu.VMEM_SHARED((NS, HALF), jnp.int32),
            pltpu.VMEM_SHARED((NS, HALF * KARY), jnp.int32),
        ],
        compiler_params=pltpu.CompilerParams(needs_layout_passes=False),
    )
    return kern(src, dst)


STAGES = (1024, 3072)
NW = NC * NS


def _sc_gather_body(base, npq, x_hbm, selT_hbm, feats_hbm, sel_v, rows_v,
                    sem):
    cid = lax.axis_index("c")
    sid = lax.axis_index("s")
    wid = sid * NC + cid
    gbase = base + wid * npq
    obase = wid * npq
    w1 = [pltpu.async_copy(selT_hbm.at[pl.ds(t * N + gbase, npq)],
                           sel_v.at[pl.ds(t * npq, npq)], sem)
          for t in range(KARY)]
    for c in w1:
        c.wait()
    w2 = [pltpu.async_copy(x_hbm.at[sel_v.at[pl.ds(t * npq, npq)]],
                           rows_v.at[pl.ds(t * npq, npq)], sem)
          for t in range(KARY)]
    for c in w2:
        c.wait()
    w3 = [pltpu.async_copy(rows_v.at[pl.ds(t * npq, npq)],
                           feats_hbm.at[t, pl.ds(obase, npq)], sem)
          for t in range(KARY)]
    for c in w3:
        c.wait()


def _sc_gather(x_pk, selT, base, count):
    npq = count // NW
    mesh = plsc.VectorSubcoreMesh(core_axis_name="c", subcore_axis_name="s")
    kern = pl.kernel(
        functools.partial(_sc_gather_body, base, npq),
        out_type=jax.ShapeDtypeStruct((KARY, count, F // 2), jnp.int32),
        mesh=mesh,
        scratch_types=[
            pltpu.VMEM((KARY * npq,), jnp.int32),
            pltpu.VMEM((KARY * npq, F // 2), jnp.int32),
            pltpu.SemaphoreType.DMA,
        ],
        compiler_params=pltpu.CompilerParams(needs_layout_passes=False),
    )
    return kern(x_pk, selT)


def _pack_body(x_ref, wih_ref, whh_ref, wout_ref, xpk_ref, wihb_ref, whhb_ref,
               woutb_ref):
    xa = lax.bitcast_convert_type(x_ref[:, :F // 2], jnp.int32)
    xb = lax.bitcast_convert_type(x_ref[:, F // 2:], jnp.int32)
    half = jnp.int32(0x8000)
    lo = ((xa + half) >> 16) & jnp.int32(0xFFFF)
    hi = (xb + half) & jnp.int32(-65536)
    xpk_ref[...] = lo | hi
    bf = jnp.bfloat16
    wihb_ref[...] = wih_ref[...].astype(bf)
    whhb_ref[...] = whh_ref[...].astype(bf)
    woutb_ref[...] = wout_ref[...].astype(bf)


def _tc_pack(x, W_ih, W_hh, W_out):
    bf = jnp.bfloat16
    return pl.pallas_call(
        _pack_body,
        out_shape=[
            jax.ShapeDtypeStruct((N, F // 2), jnp.int32),
            jax.ShapeDtypeStruct((4 * F, F), bf),
            jax.ShapeDtypeStruct((4 * F, F), bf),
            jax.ShapeDtypeStruct((2 * F, OUT), bf),
        ],
    )(x, W_ih, W_hh, W_out)


def _dot_t(a, w):
    return lax.dot_general(a, w, (((1,), (1,)), ((), ())),
                           preferred_element_type=jnp.float32)


def _tc_body(x_ref, feats_ref, len_ref, wih_ref, whh_ref, bih_ref,
             bhh_ref, wout_ref, bout_ref, out_ref):
    xb = x_ref[...]
    b = bih_ref[...] + bhh_ref[...]
    lens = len_ref[...]
    blk = xb.shape[0]
    c = jnp.zeros((blk, F), jnp.float32)
    h = jnp.zeros((blk, F), jnp.float32)
    bf = jnp.bfloat16
    for t in range(KARY):
        fi = feats_ref[t]
        ft_lo = lax.bitcast_convert_type(fi << 16, jnp.float32).astype(bf)
        ft_hi = lax.bitcast_convert_type(fi & jnp.int32(-65536),
                                         jnp.float32).astype(bf)
        gates = (_dot_t(ft_lo, wih_ref[:, :F // 2])
                 + _dot_t(ft_hi, wih_ref[:, F // 2:]) + b)
        if t > 0:
            gates = gates + _dot_t(h.astype(bf), whh_ref[...])
        i_g = gates[:, :F]
        f_g = gates[:, F:2 * F]
        g_g = gates[:, 2 * F:3 * F]
        o_g = gates[:, 3 * F:]
        c_new = jax.nn.sigmoid(f_g) * c + jax.nn.sigmoid(i_g) * jnp.tanh(g_g)
        h_new = jax.nn.sigmoid(o_g) * jnp.tanh(c_new)
        m = t < lens
        c = jnp.where(m, c_new, c)
        h = jnp.where(m, h_new, h)
    y = (jnp.dot(xb.astype(bf), wout_ref[:F],
                 preferred_element_type=jnp.float32)
         + jnp.dot(c.astype(bf), wout_ref[F:],
                   preferred_element_type=jnp.float32)
         + bout_ref[...])
    out_ref[...] = jax.nn.sigmoid(y)


def _tc_lstm(x, feats, lens, wihb, whhb, bih, bhh, woutb, bout, base, count):
    B = 512
    hb = base // B
    grid = (count // B,)
    return pl.pallas_call(
        _tc_body,
        grid=grid,
        in_specs=[
            pl.BlockSpec((B, F), lambda i: (i + hb, 0)),
            pl.BlockSpec((KARY, B, F // 2), lambda i: (0, i, 0)),
            pl.BlockSpec((B, 1), lambda i: (i + hb, 0)),
            pl.BlockSpec((4 * F, F), lambda i: (0, 0)),
            pl.BlockSpec((4 * F, F), lambda i: (0, 0)),
            pl.BlockSpec((1, 4 * F), lambda i: (0, 0)),
            pl.BlockSpec((1, 4 * F), lambda i: (0, 0)),
            pl.BlockSpec((2 * F, OUT), lambda i: (0, 0)),
            pl.BlockSpec((1, OUT), lambda i: (0, 0)),
        ],
        out_specs=pl.BlockSpec((B, OUT), lambda i: (i, 0)),
        out_shape=jax.ShapeDtypeStruct((count, OUT), jnp.float32),
        compiler_params=pltpu.CompilerParams(
            dimension_semantics=("arbitrary",)),
    )(x, feats, lens, wihb, whhb, bih, bhh, woutb, bout)


def kernel(node_feat_input, adjacency_input, indices, W_ih, W_hh, b_ih, b_hh,
           W_out, b_out):
    src = adjacency_input[:, 0]
    dst = adjacency_input[:, 1]
    x_pk, wihb, whhb, woutb = _tc_pack(node_feat_input, W_ih, W_hh, W_out)
    selT, lengths = _sc_build(src, dst)
    lens2 = lengths.reshape(N, 1)
    bih2 = b_ih.reshape(1, -1)
    bhh2 = b_hh.reshape(1, -1)
    bout2 = b_out.reshape(1, -1)
    outs = []
    base = 0
    for count in STAGES:
        feats_q = _sc_gather(x_pk, selT, base, count)
        outs.append(_tc_lstm(node_feat_input, feats_q, lens2, wihb, whhb,
                             bih2, bhh2, woutb, bout2, base, count))
        base += count
    return jnp.concatenate(outs, axis=0)


# speedup vs baseline: 1.5930x; 1.0561x over previous
import functools

import jax
import jax.numpy as jnp
from jax import lax
from jax.experimental import pallas as pl
from jax.experimental.pallas import tpu as pltpu
from jax.experimental.pallas import tpu_sc as plsc

N = 4096
E = 65536
F = 256
OUT = 128
KARY = 5

NC = 2
NS = 16
L = 16
HALF = N // NC
NPT = HALF // NS
CHUNK = E // NS


def _sc_build_body(src_hbm, dst_hbm, selT_hbm, len_hbm,
                   src_v, dst_v, cnt_v, loc_v, cnt_t, loc_t, selT, len_t,
                   counts_sh, locs_sh):
    cid = lax.axis_index("c")
    sid = lax.axis_index("s")
    lo = cid * HALF
    lanes = lax.iota(jnp.int32, L)

    pltpu.sync_copy(src_hbm.at[pl.ds(sid * CHUNK, CHUNK)], src_v)
    pltpu.sync_copy(dst_hbm.at[pl.ds(sid * CHUNK, CHUNK)], dst_v)

    @plsc.parallel_loop(0, HALF, step=L)
    def _(i):
        cnt_v[pl.ds(i, L)] = jnp.zeros((L,), jnp.int32)

    nv = CHUNK // L

    def stage1(i):
        s16 = src_v[pl.ds(i * L, L)]
        d16 = dst_v[pl.ds(i * L, L)]
        sl_raw = s16 - lo
        valid = (sl_raw >= 0) & (sl_raw < HALF)
        occ, lastm = plsc.scan_count(sl_raw, mask=valid)
        return d16, sl_raw, valid, occ, lastm

    def edge_body(i, carry):
        d16, sl_raw, valid, occ, lastm = carry
        nxt = stage1(jnp.minimum(i + 1, nv - 1))
        sl = sl_raw & (HALF - 1)
        cnt16 = plsc.load_gather(cnt_v, [sl], mask=valid)
        p = cnt16 + occ - 1
        plsc.store_scatter(loc_v, [sl * KARY + p], d16,
                           mask=valid & (p < KARY))
        plsc.store_scatter(cnt_v, [sl], cnt16 + occ, mask=valid & lastm)
        return nxt

    lax.fori_loop(0, nv, edge_body, stage1(0))

    pltpu.sync_copy(cnt_v, counts_sh.at[sid])
    pltpu.sync_copy(loc_v, locs_sh.at[sid])
    plsc.subcore_barrier()

    nb = sid * NPT
    pltpu.sync_copy(counts_sh.at[:, pl.ds(nb, NPT)], cnt_t)
    pltpu.sync_copy(locs_sh.at[:, pl.ds(nb * KARY, NPT * KARY)], loc_t)

    @plsc.parallel_loop(0, NPT * KARY, step=L)
    def _(i):
        selT[pl.ds(i, L)] = jnp.zeros((L,), jnp.int32)

    @plsc.parallel_loop(0, NPT, step=L)
    def _(j):
        acc = jnp.zeros((L,), jnp.int32)
        for w in range(NS):
            acc = acc + jnp.minimum(cnt_t[w, pl.ds(j, L)], KARY)
        len_t[pl.ds(j, L)] = jnp.minimum(acc, KARY)

    @plsc.parallel_loop(0, NPT, unroll=2)
    def _(j):
        jv = jnp.full((L,), j, jnp.int32)
        col = plsc.load_gather(cnt_t, [lanes, jv])
        cc = jnp.minimum(col, KARY)
        off = plsc.cumsum(cc) - cc
        for k in range(KARY):
            vals = plsc.load_gather(loc_t, [lanes, jv * KARY + k])
            pos = off + k
            m = (k < cc) & (pos < KARY)
            plsc.store_scatter(selT, [pos * NPT + j], vals, mask=m)

    pltpu.sync_copy(len_t, len_hbm.at[pl.ds(lo + nb, NPT)])
    for t in range(KARY):
        pltpu.sync_copy(selT.at[pl.ds(t * NPT, NPT)],
                        selT_hbm.at[pl.ds(t * N + lo + nb, NPT)])


def _sc_build(src, dst):
    mesh = plsc.VectorSubcoreMesh(core_axis_name="c", subcore_axis_name="s")
    kern = pl.kernel(
        _sc_build_body,
        out_type=[
            jax.ShapeDtypeStruct((KARY * N,), jnp.int32),
            jax.ShapeDtypeStruct((N,), jnp.int32),
        ],
        mesh=mesh,
        scratch_types=[
            pltpu.VMEM((CHUNK,), jnp.int32),
            pltpu.VMEM((CHUNK,), jnp.int32),
            pltpu.VMEM((HALF,), jnp.int32),
            pltpu.VMEM((HALF * KARY,), jnp.int32),
            pltpu.VMEM((NS, NPT), jnp.int32),
            pltpu.VMEM((NS, NPT * KARY), jnp.int32),
            pltpu.VMEM((NPT * KARY,), jnp.int32),
            pltpu.VMEM((NPT,), jnp.int32),
            pltpu.VMEM_SHARED((NS, HALF), jnp.int32),
            pltpu.VMEM_SHARED((NS, HALF * KARY), jnp.int32),
        ],
        compiler_params=pltpu.CompilerParams(needs_layout_passes=False),
    )
    return kern(src, dst)


STAGES = (1024, 3072)
NW = NC * NS


def _sc_gather_body(base, npq, x_hbm, selT_hbm, feats_hbm, sel_v, rows_v,
                    sem):
    cid = lax.axis_index("c")
    sid = lax.axis_index("s")
    wid = sid * NC + cid
    gbase = base + wid * npq
    obase = wid * npq
    w1 = [pltpu.async_copy(selT_hbm.at[pl.ds(t * N + gbase, npq)],
                           sel_v.at[pl.ds(t * npq, npq)], sem)
          for t in range(KARY)]
    for c in w1:
        c.wait()
    w2 = [pltpu.async_copy(x_hbm.at[sel_v.at[pl.ds(t * npq, npq)]],
                           rows_v.at[pl.ds(t * npq, npq)], sem)
          for t in range(KARY)]
    for c in w2:
        c.wait()
    w3 = [pltpu.async_copy(rows_v.at[pl.ds(t * npq, npq)],
                           feats_hbm.at[t, pl.ds(obase, npq)], sem)
          for t in range(KARY)]
    for c in w3:
        c.wait()


def _sc_gather(x_pk, selT, base, count):
    npq = count // NW
    mesh = plsc.VectorSubcoreMesh(core_axis_name="c", subcore_axis_name="s")
    kern = pl.kernel(
        functools.partial(_sc_gather_body, base, npq),
        out_type=jax.ShapeDtypeStruct((KARY, count, F // 2), jnp.int32),
        mesh=mesh,
        scratch_types=[
            pltpu.VMEM((KARY * npq,), jnp.int32),
            pltpu.VMEM((KARY * npq, F // 2), jnp.int32),
            pltpu.SemaphoreType.DMA,
        ],
        compiler_params=pltpu.CompilerParams(needs_layout_passes=False),
    )
    return kern(x_pk, selT)


def _pack_body(x_ref, wih_ref, whh_ref, wout_ref, xpk_ref, wihb_ref, whhb_ref,
               woutb_ref):
    xa = lax.bitcast_convert_type(x_ref[:, :F // 2], jnp.int32)
    xb = lax.bitcast_convert_type(x_ref[:, F // 2:], jnp.int32)
    half = jnp.int32(0x8000)
    lo = ((xa + half) >> 16) & jnp.int32(0xFFFF)
    hi = (xb + half) & jnp.int32(-65536)
    xpk_ref[...] = lo | hi
    bf = jnp.bfloat16
    wihb_ref[...] = wih_ref[...].astype(bf)
    whhb_ref[...] = whh_ref[...].astype(bf)
    woutb_ref[...] = wout_ref[...].astype(bf)


def _tc_pack(x, W_ih, W_hh, W_out):
    bf = jnp.bfloat16
    return pl.pallas_call(
        _pack_body,
        out_shape=[
            jax.ShapeDtypeStruct((N, F // 2), jnp.int32),
            jax.ShapeDtypeStruct((4 * F, F), bf),
            jax.ShapeDtypeStruct((4 * F, F), bf),
            jax.ShapeDtypeStruct((2 * F, OUT), bf),
        ],
    )(x, W_ih, W_hh, W_out)


def _dot_t(a, w):
    return lax.dot_general(a, w, (((1,), (1,)), ((), ())),
                           preferred_element_type=jnp.float32)


def _tc_body(x_ref, feats_ref, len_ref, wih_ref, whh_ref, bih_ref,
             bhh_ref, wout_ref, bout_ref, out_ref):
    xb = x_ref[...]
    b = bih_ref[...] + bhh_ref[...]
    lens = len_ref[...]
    blk = xb.shape[0]
    c = jnp.zeros((blk, F), jnp.float32)
    h = jnp.zeros((blk, F), jnp.float32)
    bf = jnp.bfloat16
    for t in range(KARY):
        fi = feats_ref[t]
        ft_lo = lax.bitcast_convert_type(fi << 16, jnp.float32).astype(bf)
        ft_hi = lax.bitcast_convert_type(fi & jnp.int32(-65536),
                                         jnp.float32).astype(bf)
        gates = (_dot_t(ft_lo, wih_ref[:, :F // 2])
                 + _dot_t(ft_hi, wih_ref[:, F // 2:]) + b)
        if t > 0:
            gates = gates + _dot_t(h.astype(bf), whh_ref[...])
        i_g = gates[:, :F]
        f_g = gates[:, F:2 * F]
        g_g = gates[:, 2 * F:3 * F]
        o_g = gates[:, 3 * F:]
        c_new = jax.nn.sigmoid(f_g) * c + jax.nn.sigmoid(i_g) * jnp.tanh(g_g)
        h_new = jax.nn.sigmoid(o_g) * jnp.tanh(c_new)
        m = t < lens
        c = jnp.where(m, c_new, c)
        h = jnp.where(m, h_new, h)
    y = (jnp.dot(xb.astype(bf), wout_ref[:F],
                 preferred_element_type=jnp.float32)
         + jnp.dot(c.astype(bf), wout_ref[F:],
                   preferred_element_type=jnp.float32)
         + bout_ref[...])
    out_ref[...] = jax.nn.sigmoid(y)


def _tc_lstm(x, feats, lens, wihb, whhb, bih, bhh, woutb, bout, base, count):
    B = 512
    hb = base // B
    grid = (count // B,)
    return pl.pallas_call(
        _tc_body,
        grid=grid,
        in_specs=[
            pl.BlockSpec((B, F), lambda i: (i + hb, 0)),
            pl.BlockSpec((KARY, B, F // 2), lambda i: (0, i, 0)),
            pl.BlockSpec((B, 1), lambda i: (i + hb, 0)),
            pl.BlockSpec((4 * F, F), lambda i: (0, 0)),
            pl.BlockSpec((4 * F, F), lambda i: (0, 0)),
            pl.BlockSpec((1, 4 * F), lambda i: (0, 0)),
            pl.BlockSpec((1, 4 * F), lambda i: (0, 0)),
            pl.BlockSpec((2 * F, OUT), lambda i: (0, 0)),
            pl.BlockSpec((1, OUT), lambda i: (0, 0)),
        ],
        out_specs=pl.BlockSpec((B, OUT), lambda i: (i, 0)),
        out_shape=jax.ShapeDtypeStruct((count, OUT), jnp.float32),
        compiler_params=pltpu.CompilerParams(
            dimension_semantics=("arbitrary",)),
    )(x, feats, lens, wihb, whhb, bih, bhh, woutb, bout)


def kernel(node_feat_input, adjacency_input, indices, W_ih, W_hh, b_ih, b_hh,
           W_out, b_out):
    src = adjacency_input[:, 0]
    dst = adjacency_input[:, 1]
    x_pk, wihb, whhb, woutb = _tc_pack(node_feat_input, W_ih, W_hh, W_out)
    selT, lengths = _sc_build(src, dst)
    lens2 = lengths.reshape(N, 1)
    bih2 = b_ih.reshape(1, -1)
    bhh2 = b_hh.reshape(1, -1)
    bout2 = b_out.reshape(1, -1)
    outs = []
    base = 0
    for count in STAGES:
        feats_q = _sc_gather(x_pk, selT, base, count)
        outs.append(_tc_lstm(node_feat_input, feats_q, lens2, wihb, whhb,
                             bih2, bhh2, woutb, bout2, base, count))
        base += count
    return jnp.concatenate(outs, axis=0)
